# Initial kernel scaffold; baseline (speedup 1.0000x reference)
import jax
import jax.numpy as jnp
from jax.experimental import pallas as pl


def kernel(trg_token_ids_batch):
    raise NotImplementedError("write your pallas kernel here")


# speedup vs baseline: 8.5199x; 8.5199x over previous
import jax
import jax.numpy as jnp
from jax.experimental import pallas as pl

SMOOTHING_VALUE = 0.1
CONFIDENCE_VALUE = 1.0 - SMOOTHING_VALUE
PAD_TOKEN_ID = 0
TRG_VOCAB_SIZE = 32000

BR = 256
BC = 6400


def _smooth_kernel(tok_ref, out_ref):
    j = pl.program_id(1)
    fill = SMOOTHING_VALUE / (TRG_VOCAB_SIZE - 2)
    col = jax.lax.broadcasted_iota(jnp.int32, (BR, BC), 1) + j * BC
    t = tok_ref[:, 0][:, None]
    val = jnp.where(col == t, CONFIDENCE_VALUE, fill)
    val = jnp.where((col == PAD_TOKEN_ID) | (t == PAD_TOKEN_ID), 0.0, val)
    out_ref[...] = val


def kernel(trg_token_ids_batch):
    b = trg_token_ids_batch.shape[0]
    tok = trg_token_ids_batch.astype(jnp.int32)
    return pl.pallas_call(
        _smooth_kernel,
        grid=(b // BR, TRG_VOCAB_SIZE // BC),
        in_specs=[pl.BlockSpec((BR, 1), lambda i, j: (i, 0))],
        out_specs=pl.BlockSpec((BR, BC), lambda i, j: (i, j)),
        out_shape=jax.ShapeDtypeStruct((b, TRG_VOCAB_SIZE), jnp.float32),
    )(tok)
